# baseline (speedup 1.0000x reference)
import functools

import jax
import jax.numpy as jnp
from jax import lax
from jax.experimental import pallas as pl
from jax.experimental.pallas import tpu as pltpu
from jax.experimental.pallas import tpu_sc as plsc

_N_E = 8192
_E_DIM = 256
_BETA = 0.25

_BM = 256
_BK = 2736
_NKC = 3
_N_E_PAD = _BK * _NKC
_PAD_VAL = 3.0e4


def _dist_argmin_body(x_ref, e_ref, xs_ref, en_ref,
                      idx_ref, loss_ref, rmin_ref, rval_ref, rarg_ref):
    i = pl.program_id(0)
    j = pl.program_id(1)
    x_blk = x_ref[...]
    xs_row = xs_ref[...]
    e_blk = e_ref[...]
    en = en_ref[...]
    mt = lax.dot_general(
        e_blk, x_blk, (((1,), (0,)), ((), ())),
        preferred_element_type=jnp.float32,
        precision=lax.Precision.DEFAULT)
    s = (xs_row + en) - 2.0 * mt
    minv = jnp.min(s, axis=0, keepdims=True)
    iota = lax.broadcasted_iota(jnp.int32, (_BK, _BM), 0) + j * _BK
    carg = jnp.min(jnp.where(s == minv, iota, jnp.int32(2 ** 30)),
                   axis=0, keepdims=True)
    minb = minv.astype(jnp.bfloat16).astype(jnp.float32)

    @pl.when(j == 0)
    def _():
        rmin_ref[...] = minb
        rval_ref[...] = minv
        rarg_ref[...] = carg

    @pl.when(j > 0)
    def _():
        take = minv < rmin_ref[...]
        rarg_ref[...] = jnp.where(take, carg, rarg_ref[...])
        rval_ref[...] = jnp.where(take, minv, rval_ref[...])
        rmin_ref[...] = jnp.where(take, minb, rmin_ref[...])

    @pl.when(j == _NKC - 1)
    def _():
        idx_ref[0, 0, :] = rarg_ref[0, :]
        part = jnp.sum(rval_ref[...])

        @pl.when(i == 0)
        def _():
            loss_ref[0, 0] = part

        @pl.when(i > 0)
        def _():
            loss_ref[0, 0] = loss_ref[0, 0] + part


def _dist_argmin(latent_t, emb_pad, xs_row, en_col):
    m = latent_t.shape[1]
    nm = m // _BM
    return pl.pallas_call(
        _dist_argmin_body,
        grid=(nm, _NKC),
        in_specs=[
            pl.BlockSpec((_E_DIM, _BM), lambda i, j: (0, i)),
            pl.BlockSpec((_BK, _E_DIM), lambda i, j: (j, 0)),
            pl.BlockSpec((1, _BM), lambda i, j: (0, i)),
            pl.BlockSpec((_BK, 1), lambda i, j: (j, 0)),
        ],
        out_specs=[
            pl.BlockSpec((1, 1, _BM), lambda i, j: (i, 0, 0)),
            pl.BlockSpec((1, 1), lambda i, j: (0, 0),
                         memory_space=pltpu.SMEM),
        ],
        out_shape=[
            jax.ShapeDtypeStruct((nm, 1, _BM), jnp.int32),
            jax.ShapeDtypeStruct((1, 1), jnp.float32),
        ],
        scratch_shapes=[
            pltpu.VMEM((1, _BM), jnp.float32),
            pltpu.VMEM((1, _BM), jnp.float32),
            pltpu.VMEM((1, _BM), jnp.int32),
        ],
        compiler_params=pltpu.CompilerParams(
            dimension_semantics=("arbitrary", "arbitrary")),
    )(latent_t, emb_pad, xs_row, en_col)


def _sc_gather(embedding, indices):
    b = indices.shape[0]
    info = plsc.get_sparse_core_info()
    nc, ns = info.num_cores, info.num_subcores
    nw = nc * ns
    b_per_w = b // nw
    ch = 128
    n_ch = b_per_w // ch
    mesh = plsc.VectorSubcoreMesh(core_axis_name="c", subcore_axis_name="s")

    @functools.partial(
        pl.kernel,
        out_type=jax.ShapeDtypeStruct((b, _E_DIM), jnp.float32),
        mesh=mesh,
        scratch_types=[
            pltpu.VMEM((ch,), jnp.int32),
            pltpu.VMEM((ch,), jnp.int32),
            pltpu.VMEM((ch, _E_DIM), jnp.float32),
            pltpu.VMEM((ch, _E_DIM), jnp.float32),
            pltpu.SemaphoreType.DMA,
            pltpu.SemaphoreType.DMA,
        ],
    )
    def gather_k(emb_hbm, idx_hbm, out_hbm, idx0, idx1, rows0, rows1,
                 sem0, sem1):
        wid = lax.axis_index("s") * nc + lax.axis_index("c")
        base = wid * b_per_w
        idxs = (idx0, idx1)
        bufs = (rows0, rows1)
        sems = (sem0, sem1)
        copies = [None, None]
        for c in range(n_ch):
            slot = c % 2
            pltpu.sync_copy(idx_hbm.at[pl.ds(base + c * ch, ch)], idxs[slot])
            copies[slot] = pltpu.async_copy(
                emb_hbm.at[idxs[slot]], bufs[slot], sems[slot])
            if c > 0:
                prev = 1 - slot
                copies[prev].wait()
                pltpu.sync_copy(bufs[prev],
                                out_hbm.at[pl.ds(base + (c - 1) * ch, ch)])
        last = (n_ch - 1) % 2
        copies[last].wait()
        pltpu.sync_copy(bufs[last],
                        out_hbm.at[pl.ds(base + (n_ch - 1) * ch, ch)])

    return gather_k(embedding, indices)


def kernel(x, embedding):
    latent = x.reshape(-1, _E_DIM)
    m = latent.shape[0]
    xs_row = jnp.sum(latent ** 2, axis=1, keepdims=True).reshape(1, m)
    en_col = jnp.sum(embedding ** 2, axis=1, keepdims=True)
    emb_pad = jnp.pad(embedding, ((0, _N_E_PAD - _N_E), (0, 0)),
                      constant_values=_PAD_VAL)
    en_pad = jnp.pad(en_col, ((0, _N_E_PAD - _N_E), (0, 0)),
                     constant_values=_PAD_VAL * _PAD_VAL * _E_DIM)
    idx3, loss_sum = _dist_argmin(latent.T, emb_pad, xs_row, en_pad)
    indices = idx3.reshape(-1)
    x_q = _sc_gather(embedding, indices)
    loss = loss_sum[0, 0] * (_BETA / (m * _E_DIM))
    return (x_q.reshape(x.shape), loss, indices.reshape(x.shape[:-1]))


# speedup vs baseline: 1.2208x; 1.2208x over previous
import functools

import jax
import jax.numpy as jnp
from jax import lax
from jax.experimental import pallas as pl
from jax.experimental.pallas import tpu as pltpu
from jax.experimental.pallas import tpu_sc as plsc

_N_E = 8192
_E_DIM = 256
_BETA = 0.25

_BM = 256
_BK = 2736
_NKC = 3
_N_E_PAD = _BK * _NKC
_PAD_VAL = 3.0e4


def _dist_argmin_body(x_ref, e_ref, xs_ref, en_ref,
                      idx_ref, loss_ref, rmin_ref, rval_ref, rarg_ref):
    j = pl.program_id(0)
    i = pl.program_id(1)
    x_blk = x_ref[...]
    xs_row = xs_ref[...]
    e_blk = e_ref[...]
    en = en_ref[...]
    mt = lax.dot_general(
        e_blk, x_blk, (((1,), (0,)), ((), ())),
        preferred_element_type=jnp.float32,
        precision=lax.Precision.DEFAULT)
    s = (xs_row + en) - 2.0 * mt
    minv = jnp.min(s, axis=0, keepdims=True)
    iota = lax.broadcasted_iota(jnp.int32, (_BK, _BM), 0) + j * _BK
    carg = jnp.min(jnp.where(s == minv, iota, jnp.int32(2 ** 30)),
                   axis=0, keepdims=True)
    minb = minv.astype(jnp.bfloat16).astype(jnp.float32)
    sl = pl.ds(i * _BM, _BM)

    @pl.when(j == 0)
    def _():
        rmin_ref[:, sl] = minb
        rval_ref[:, sl] = minv
        rarg_ref[:, sl] = carg

    @pl.when(j > 0)
    def _():
        take = minv < rmin_ref[:, sl]
        rarg_ref[:, sl] = jnp.where(take, carg, rarg_ref[:, sl])
        rval_ref[:, sl] = jnp.where(take, minv, rval_ref[:, sl])
        rmin_ref[:, sl] = jnp.where(take, minb, rmin_ref[:, sl])

    @pl.when(j == _NKC - 1)
    def _():
        idx_ref[0, 0, :] = rarg_ref[0, sl]
        part = jnp.sum(rval_ref[:, sl])

        @pl.when(i == 0)
        def _():
            loss_ref[0, 0] = part

        @pl.when(i > 0)
        def _():
            loss_ref[0, 0] = loss_ref[0, 0] + part


def _dist_argmin(latent_t, emb_pad, xs_row, en_col):
    m = latent_t.shape[1]
    nm = m // _BM
    return pl.pallas_call(
        _dist_argmin_body,
        grid=(_NKC, nm),
        in_specs=[
            pl.BlockSpec((_E_DIM, _BM), lambda j, i: (0, i)),
            pl.BlockSpec((_BK, _E_DIM), lambda j, i: (j, 0)),
            pl.BlockSpec((1, _BM), lambda j, i: (0, i)),
            pl.BlockSpec((_BK, 1), lambda j, i: (j, 0)),
        ],
        out_specs=[
            pl.BlockSpec((1, 1, _BM), lambda j, i: (i, 0, 0)),
            pl.BlockSpec((1, 1), lambda j, i: (0, 0),
                         memory_space=pltpu.SMEM),
        ],
        out_shape=[
            jax.ShapeDtypeStruct((nm, 1, _BM), jnp.int32),
            jax.ShapeDtypeStruct((1, 1), jnp.float32),
        ],
        scratch_shapes=[
            pltpu.VMEM((1, 16384), jnp.float32),
            pltpu.VMEM((1, 16384), jnp.float32),
            pltpu.VMEM((1, 16384), jnp.int32),
        ],
        compiler_params=pltpu.CompilerParams(
            dimension_semantics=("arbitrary", "arbitrary")),
    )(latent_t, emb_pad, xs_row, en_col)


def _sc_gather(embedding, indices):
    b = indices.shape[0]
    info = plsc.get_sparse_core_info()
    nc, ns = info.num_cores, info.num_subcores
    nw = nc * ns
    b_per_w = b // nw
    ch = 128
    n_ch = b_per_w // ch
    mesh = plsc.VectorSubcoreMesh(core_axis_name="c", subcore_axis_name="s")

    @functools.partial(
        pl.kernel,
        out_type=jax.ShapeDtypeStruct((b, _E_DIM), jnp.float32),
        mesh=mesh,
        scratch_types=[
            pltpu.VMEM((ch,), jnp.int32),
            pltpu.VMEM((ch,), jnp.int32),
            pltpu.VMEM((ch, _E_DIM), jnp.float32),
            pltpu.VMEM((ch, _E_DIM), jnp.float32),
            pltpu.SemaphoreType.DMA,
            pltpu.SemaphoreType.DMA,
        ],
    )
    def gather_k(emb_hbm, idx_hbm, out_hbm, idx0, idx1, rows0, rows1,
                 sem0, sem1):
        wid = lax.axis_index("s") * nc + lax.axis_index("c")
        base = wid * b_per_w
        idxs = (idx0, idx1)
        bufs = (rows0, rows1)
        sems = (sem0, sem1)
        copies = [None, None]
        for c in range(n_ch):
            slot = c % 2
            pltpu.sync_copy(idx_hbm.at[pl.ds(base + c * ch, ch)], idxs[slot])
            copies[slot] = pltpu.async_copy(
                emb_hbm.at[idxs[slot]], bufs[slot], sems[slot])
            if c > 0:
                prev = 1 - slot
                copies[prev].wait()
                pltpu.sync_copy(bufs[prev],
                                out_hbm.at[pl.ds(base + (c - 1) * ch, ch)])
        last = (n_ch - 1) % 2
        copies[last].wait()
        pltpu.sync_copy(bufs[last],
                        out_hbm.at[pl.ds(base + (n_ch - 1) * ch, ch)])

    return gather_k(embedding, indices)


def kernel(x, embedding):
    latent = x.reshape(-1, _E_DIM)
    m = latent.shape[0]
    xs_row = jnp.sum(latent ** 2, axis=1, keepdims=True).reshape(1, m)
    en_col = jnp.sum(embedding ** 2, axis=1, keepdims=True)
    emb_pad = jnp.pad(embedding, ((0, _N_E_PAD - _N_E), (0, 0)),
                      constant_values=_PAD_VAL)
    en_pad = jnp.pad(en_col, ((0, _N_E_PAD - _N_E), (0, 0)),
                     constant_values=_PAD_VAL * _PAD_VAL * _E_DIM)
    idx3, loss_sum = _dist_argmin(latent.T, emb_pad, xs_row, en_pad)
    indices = idx3.reshape(-1)
    x_q = _sc_gather(embedding, indices)
    loss = loss_sum[0, 0] * (_BETA / (m * _E_DIM))
    return (x_q.reshape(x.shape), loss, indices.reshape(x.shape[:-1]))


# speedup vs baseline: 1.2414x; 1.0169x over previous
import functools

import jax
import jax.numpy as jnp
from jax import lax
from jax.experimental import pallas as pl
from jax.experimental.pallas import tpu as pltpu
from jax.experimental.pallas import tpu_sc as plsc

_N_E = 8192
_E_DIM = 256
_BETA = 0.25

_BM = 256
_BK = 2736
_NKC = 3
_N_E_PAD = _BK * _NKC
_PAD_VAL = 3.0e4


def _dist_argmin_body(x_ref, e_ref, xs_ref, en_ref,
                      idx_ref, loss_ref, rmin_ref, rval_ref, rarg_ref):
    j = pl.program_id(0)
    i = pl.program_id(1)
    x_blk = x_ref[...]
    xs_row = xs_ref[...]
    e_blk = e_ref[...]
    en = en_ref[...]
    mt = lax.dot_general(
        e_blk, x_blk, (((1,), (0,)), ((), ())),
        preferred_element_type=jnp.float32,
        precision=lax.Precision.DEFAULT)
    s = (xs_row + en) - 2.0 * mt
    minv = jnp.min(s, axis=0, keepdims=True)
    iota = lax.broadcasted_iota(jnp.int32, (_BK, _BM), 0) + j * _BK
    carg = jnp.min(jnp.where(s == minv, iota, jnp.int32(2 ** 30)),
                   axis=0, keepdims=True)
    minb = minv.astype(jnp.bfloat16).astype(jnp.float32)
    sl = pl.ds(i * _BM, _BM)

    @pl.when(j == 0)
    def _():
        rmin_ref[:, sl] = minb
        rval_ref[:, sl] = minv
        rarg_ref[:, sl] = carg

    @pl.when(j > 0)
    def _():
        take = minv < rmin_ref[:, sl]
        rarg_ref[:, sl] = jnp.where(take, carg, rarg_ref[:, sl])
        rval_ref[:, sl] = jnp.where(take, minv, rval_ref[:, sl])
        rmin_ref[:, sl] = jnp.where(take, minb, rmin_ref[:, sl])

    @pl.when(j == _NKC - 1)
    def _():
        idx_ref[0, 0, :] = rarg_ref[0, sl]
        part = jnp.sum(rval_ref[:, sl])

        @pl.when(i == 0)
        def _():
            loss_ref[0, 0] = part

        @pl.when(i > 0)
        def _():
            loss_ref[0, 0] = loss_ref[0, 0] + part


def _dist_argmin(latent_t, emb_pad, xs_row, en_col):
    m = latent_t.shape[1]
    nm = m // _BM
    return pl.pallas_call(
        _dist_argmin_body,
        grid=(_NKC, nm),
        in_specs=[
            pl.BlockSpec((_E_DIM, _BM), lambda j, i: (0, i)),
            pl.BlockSpec((_BK, _E_DIM), lambda j, i: (j, 0)),
            pl.BlockSpec((1, _BM), lambda j, i: (0, i)),
            pl.BlockSpec((_BK, 1), lambda j, i: (j, 0)),
        ],
        out_specs=[
            pl.BlockSpec((1, 1, _BM), lambda j, i: (i, 0, 0)),
            pl.BlockSpec((1, 1), lambda j, i: (0, 0),
                         memory_space=pltpu.SMEM),
        ],
        out_shape=[
            jax.ShapeDtypeStruct((nm, 1, _BM), jnp.int32),
            jax.ShapeDtypeStruct((1, 1), jnp.float32),
        ],
        scratch_shapes=[
            pltpu.VMEM((1, 16384), jnp.float32),
            pltpu.VMEM((1, 16384), jnp.float32),
            pltpu.VMEM((1, 16384), jnp.int32),
        ],
        compiler_params=pltpu.CompilerParams(
            dimension_semantics=("arbitrary", "arbitrary")),
    )(latent_t, emb_pad, xs_row, en_col)


def _sc_gather(embedding, indices):
    b = indices.shape[0]
    info = plsc.get_sparse_core_info()
    nc, ns = info.num_cores, info.num_subcores
    nw = nc * ns
    b_per_w = b // nw
    ch = 128
    n_ch = b_per_w // ch
    mesh = plsc.VectorSubcoreMesh(core_axis_name="c", subcore_axis_name="s")

    @functools.partial(
        pl.kernel,
        out_type=jax.ShapeDtypeStruct((b, _E_DIM), jnp.float32),
        mesh=mesh,
        scratch_types=[
            pltpu.VMEM((ch,), jnp.int32),
            pltpu.VMEM((ch,), jnp.int32),
            pltpu.VMEM((ch, _E_DIM), jnp.float32),
            pltpu.VMEM((ch, _E_DIM), jnp.float32),
            pltpu.SemaphoreType.DMA,
            pltpu.SemaphoreType.DMA,
        ],
    )
    def gather_k(emb_hbm, idx_hbm, out_hbm, idx0, idx1, rows0, rows1,
                 sem0, sem1):
        wid = lax.axis_index("s") * nc + lax.axis_index("c")
        base = wid * b_per_w
        idxs = (idx0, idx1)
        bufs = (rows0, rows1)
        sems = (sem0, sem1)
        copies = [None, None]
        for c in range(n_ch):
            slot = c % 2
            pltpu.sync_copy(idx_hbm.at[pl.ds(base + c * ch, ch)], idxs[slot])
            copies[slot] = pltpu.async_copy(
                emb_hbm.at[idxs[slot]], bufs[slot], sems[slot])
            if c > 0:
                prev = 1 - slot
                copies[prev].wait()
                pltpu.sync_copy(bufs[prev],
                                out_hbm.at[pl.ds(base + (c - 1) * ch, ch)])
        last = (n_ch - 1) % 2
        copies[last].wait()
        pltpu.sync_copy(bufs[last],
                        out_hbm.at[pl.ds(base + (n_ch - 1) * ch, ch)])

    return gather_k(embedding, indices)


def kernel(x, embedding):
    latent = x.reshape(-1, _E_DIM)
    m = latent.shape[0]
    xs_row = jnp.sum(latent ** 2, axis=1, keepdims=True).reshape(1, m)
    en_col = jnp.sum(embedding ** 2, axis=1, keepdims=True)
    en_pad = jnp.pad(en_col, ((0, _N_E_PAD - _N_E), (0, 0)),
                     constant_values=_PAD_VAL * _PAD_VAL * _E_DIM)
    idx3, loss_sum = _dist_argmin(latent.T, embedding, xs_row, en_pad)
    indices = idx3.reshape(-1)
    x_q = _sc_gather(embedding, indices)
    loss = loss_sum[0, 0] * (_BETA / (m * _E_DIM))
    return (x_q.reshape(x.shape), loss, indices.reshape(x.shape[:-1]))


# speedup vs baseline: 1.2619x; 1.0164x over previous
import functools

import jax
import jax.numpy as jnp
from jax import lax
from jax.experimental import pallas as pl
from jax.experimental.pallas import tpu as pltpu
from jax.experimental.pallas import tpu_sc as plsc

_N_E = 8192
_E_DIM = 256
_BETA = 0.25

_BM = 256
_BK = 2736
_NKC = 3
_N_E_PAD = _BK * _NKC
_PAD_VAL = 3.0e4


def _dist_argmin_body(x_ref, e_ref, xs_ref, en_ref,
                      idx_ref, loss_ref, rmin_ref, rval_ref, rarg_ref):
    j = pl.program_id(0)
    i = pl.program_id(1)
    x_blk = x_ref[...].T
    xs_row = xs_ref[...]
    e_blk = e_ref[...]
    en = en_ref[...]
    mt = lax.dot_general(
        e_blk, x_blk, (((1,), (0,)), ((), ())),
        preferred_element_type=jnp.float32,
        precision=lax.Precision.DEFAULT)
    s = (xs_row + en) - 2.0 * mt
    minv = jnp.min(s, axis=0, keepdims=True)
    iota = lax.broadcasted_iota(jnp.int32, (_BK, _BM), 0) + j * _BK
    carg = jnp.min(jnp.where(s == minv, iota, jnp.int32(2 ** 30)),
                   axis=0, keepdims=True)
    minb = minv.astype(jnp.bfloat16).astype(jnp.float32)
    sl = pl.ds(i * _BM, _BM)

    @pl.when(j == 0)
    def _():
        rmin_ref[:, sl] = minb
        rval_ref[:, sl] = minv
        rarg_ref[:, sl] = carg

    @pl.when(j > 0)
    def _():
        take = minv < rmin_ref[:, sl]
        rarg_ref[:, sl] = jnp.where(take, carg, rarg_ref[:, sl])
        rval_ref[:, sl] = jnp.where(take, minv, rval_ref[:, sl])
        rmin_ref[:, sl] = jnp.where(take, minb, rmin_ref[:, sl])

    @pl.when(j == _NKC - 1)
    def _():
        idx_ref[0, 0, :] = rarg_ref[0, sl]
        part = jnp.sum(rval_ref[:, sl])

        @pl.when(i == 0)
        def _():
            loss_ref[0, 0] = part

        @pl.when(i > 0)
        def _():
            loss_ref[0, 0] = loss_ref[0, 0] + part


def _dist_argmin(latent, emb_pad, xs_row, en_col):
    m = latent.shape[0]
    nm = m // _BM
    return pl.pallas_call(
        _dist_argmin_body,
        grid=(_NKC, nm),
        in_specs=[
            pl.BlockSpec((_BM, _E_DIM), lambda j, i: (i, 0)),
            pl.BlockSpec((_BK, _E_DIM), lambda j, i: (j, 0)),
            pl.BlockSpec((1, _BM), lambda j, i: (0, i)),
            pl.BlockSpec((_BK, 1), lambda j, i: (j, 0)),
        ],
        out_specs=[
            pl.BlockSpec((1, 1, _BM), lambda j, i: (i, 0, 0)),
            pl.BlockSpec((1, 1), lambda j, i: (0, 0),
                         memory_space=pltpu.SMEM),
        ],
        out_shape=[
            jax.ShapeDtypeStruct((nm, 1, _BM), jnp.int32),
            jax.ShapeDtypeStruct((1, 1), jnp.float32),
        ],
        scratch_shapes=[
            pltpu.VMEM((1, 16384), jnp.float32),
            pltpu.VMEM((1, 16384), jnp.float32),
            pltpu.VMEM((1, 16384), jnp.int32),
        ],
        compiler_params=pltpu.CompilerParams(
            dimension_semantics=("arbitrary", "arbitrary")),
    )(latent, emb_pad, xs_row, en_col)


def _sc_gather(embedding, indices):
    b = indices.shape[0]
    info = plsc.get_sparse_core_info()
    nc, ns = info.num_cores, info.num_subcores
    nw = nc * ns
    b_per_w = b // nw
    ch = 128
    n_ch = b_per_w // ch
    mesh = plsc.VectorSubcoreMesh(core_axis_name="c", subcore_axis_name="s")

    @functools.partial(
        pl.kernel,
        out_type=jax.ShapeDtypeStruct((b, _E_DIM), jnp.float32),
        mesh=mesh,
        scratch_types=[
            pltpu.VMEM((ch,), jnp.int32),
            pltpu.VMEM((ch,), jnp.int32),
            pltpu.VMEM((ch, _E_DIM), jnp.float32),
            pltpu.VMEM((ch, _E_DIM), jnp.float32),
            pltpu.SemaphoreType.DMA,
            pltpu.SemaphoreType.DMA,
        ],
    )
    def gather_k(emb_hbm, idx_hbm, out_hbm, idx0, idx1, rows0, rows1,
                 sem0, sem1):
        wid = lax.axis_index("s") * nc + lax.axis_index("c")
        base = wid * b_per_w
        idxs = (idx0, idx1)
        bufs = (rows0, rows1)
        sems = (sem0, sem1)
        copies = [None, None]
        for c in range(n_ch):
            slot = c % 2
            pltpu.sync_copy(idx_hbm.at[pl.ds(base + c * ch, ch)], idxs[slot])
            copies[slot] = pltpu.async_copy(
                emb_hbm.at[idxs[slot]], bufs[slot], sems[slot])
            if c > 0:
                prev = 1 - slot
                copies[prev].wait()
                pltpu.sync_copy(bufs[prev],
                                out_hbm.at[pl.ds(base + (c - 1) * ch, ch)])
        last = (n_ch - 1) % 2
        copies[last].wait()
        pltpu.sync_copy(bufs[last],
                        out_hbm.at[pl.ds(base + (n_ch - 1) * ch, ch)])

    return gather_k(embedding, indices)


def kernel(x, embedding):
    latent = x.reshape(-1, _E_DIM)
    m = latent.shape[0]
    xs_row = jnp.sum(latent ** 2, axis=1, keepdims=True).reshape(1, m)
    en_col = jnp.sum(embedding ** 2, axis=1, keepdims=True)
    en_pad = jnp.pad(en_col, ((0, _N_E_PAD - _N_E), (0, 0)),
                     constant_values=_PAD_VAL * _PAD_VAL * _E_DIM)
    idx3, loss_sum = _dist_argmin(latent, embedding, xs_row, en_pad)
    indices = idx3.reshape(-1)
    x_q = _sc_gather(embedding, indices)
    loss = loss_sum[0, 0] * (_BETA / (m * _E_DIM))
    return (x_q.reshape(x.shape), loss, indices.reshape(x.shape[:-1]))


# speedup vs baseline: 1.3822x; 1.0954x over previous
import functools

import jax
import jax.numpy as jnp
from jax import lax
from jax.experimental import pallas as pl
from jax.experimental.pallas import tpu as pltpu
from jax.experimental.pallas import tpu_sc as plsc

_N_E = 8192
_E_DIM = 256
_BETA = 0.25

_BM = 512
_BK = 2736
_NKC = 3
_N_E_PAD = _BK * _NKC
_PAD_VAL = 3.0e4


def _dist_argmin_body(x_ref, e_ref, xs_ref, en_ref,
                      idx_ref, loss_ref, rmin_ref, rval_ref, rarg_ref):
    j = pl.program_id(0)
    i = pl.program_id(1)
    x_blk = x_ref[...].T
    xs_row = xs_ref[...]
    e_blk = e_ref[...]
    en = en_ref[...]
    mt = lax.dot_general(
        e_blk, x_blk, (((1,), (0,)), ((), ())),
        preferred_element_type=jnp.float32,
        precision=lax.Precision.DEFAULT)
    s = (xs_row + en) - 2.0 * mt
    minv = jnp.min(s, axis=0, keepdims=True)
    iota = lax.broadcasted_iota(jnp.int32, (_BK, _BM), 0) + j * _BK
    carg = jnp.min(jnp.where(s == minv, iota, jnp.int32(2 ** 30)),
                   axis=0, keepdims=True)
    minb = minv.astype(jnp.bfloat16).astype(jnp.float32)
    sl = pl.ds(i * _BM, _BM)

    @pl.when(j == 0)
    def _():
        rmin_ref[:, sl] = minb
        rval_ref[:, sl] = minv
        rarg_ref[:, sl] = carg

    @pl.when(j > 0)
    def _():
        take = minv < rmin_ref[:, sl]
        rarg_ref[:, sl] = jnp.where(take, carg, rarg_ref[:, sl])
        rval_ref[:, sl] = jnp.where(take, minv, rval_ref[:, sl])
        rmin_ref[:, sl] = jnp.where(take, minb, rmin_ref[:, sl])

    @pl.when(j == _NKC - 1)
    def _():
        idx_ref[0, 0, :] = rarg_ref[0, sl]
        part = jnp.sum(rval_ref[:, sl])

        @pl.when(i == 0)
        def _():
            loss_ref[0, 0] = part

        @pl.when(i > 0)
        def _():
            loss_ref[0, 0] = loss_ref[0, 0] + part


def _dist_argmin(latent, emb_pad, xs_row, en_col):
    m = latent.shape[0]
    nm = m // _BM
    return pl.pallas_call(
        _dist_argmin_body,
        grid=(_NKC, nm),
        in_specs=[
            pl.BlockSpec((_BM, _E_DIM), lambda j, i: (i, 0)),
            pl.BlockSpec((_BK, _E_DIM), lambda j, i: (j, 0)),
            pl.BlockSpec((1, _BM), lambda j, i: (0, i)),
            pl.BlockSpec((_BK, 1), lambda j, i: (j, 0)),
        ],
        out_specs=[
            pl.BlockSpec((1, 1, _BM), lambda j, i: (i, 0, 0)),
            pl.BlockSpec((1, 1), lambda j, i: (0, 0),
                         memory_space=pltpu.SMEM),
        ],
        out_shape=[
            jax.ShapeDtypeStruct((nm, 1, _BM), jnp.int32),
            jax.ShapeDtypeStruct((1, 1), jnp.float32),
        ],
        scratch_shapes=[
            pltpu.VMEM((1, 16384), jnp.float32),
            pltpu.VMEM((1, 16384), jnp.float32),
            pltpu.VMEM((1, 16384), jnp.int32),
        ],
        compiler_params=pltpu.CompilerParams(
            dimension_semantics=("arbitrary", "arbitrary")),
    )(latent, emb_pad, xs_row, en_col)


def _sc_gather(embedding, indices):
    b = indices.shape[0]
    info = plsc.get_sparse_core_info()
    nc, ns = info.num_cores, info.num_subcores
    nw = nc * ns
    b_per_w = b // nw
    ch = 128
    n_ch = b_per_w // ch
    mesh = plsc.VectorSubcoreMesh(core_axis_name="c", subcore_axis_name="s")

    @functools.partial(
        pl.kernel,
        out_type=jax.ShapeDtypeStruct((b, _E_DIM), jnp.float32),
        mesh=mesh,
        scratch_types=[
            pltpu.VMEM((ch,), jnp.int32),
            pltpu.VMEM((ch,), jnp.int32),
            pltpu.VMEM((ch, _E_DIM), jnp.float32),
            pltpu.VMEM((ch, _E_DIM), jnp.float32),
            pltpu.SemaphoreType.DMA,
            pltpu.SemaphoreType.DMA,
        ],
    )
    def gather_k(emb_hbm, idx_hbm, out_hbm, idx0, idx1, rows0, rows1,
                 sem0, sem1):
        wid = lax.axis_index("s") * nc + lax.axis_index("c")
        base = wid * b_per_w
        idxs = (idx0, idx1)
        bufs = (rows0, rows1)
        sems = (sem0, sem1)
        copies = [None, None]
        for c in range(n_ch):
            slot = c % 2
            pltpu.sync_copy(idx_hbm.at[pl.ds(base + c * ch, ch)], idxs[slot])
            copies[slot] = pltpu.async_copy(
                emb_hbm.at[idxs[slot]], bufs[slot], sems[slot])
            if c > 0:
                prev = 1 - slot
                copies[prev].wait()
                pltpu.sync_copy(bufs[prev],
                                out_hbm.at[pl.ds(base + (c - 1) * ch, ch)])
        last = (n_ch - 1) % 2
        copies[last].wait()
        pltpu.sync_copy(bufs[last],
                        out_hbm.at[pl.ds(base + (n_ch - 1) * ch, ch)])

    return gather_k(embedding, indices)


def kernel(x, embedding):
    latent = x.reshape(-1, _E_DIM)
    m = latent.shape[0]
    xs_row = jnp.sum(latent ** 2, axis=1, keepdims=True).reshape(1, m)
    en_col = jnp.sum(embedding ** 2, axis=1, keepdims=True)
    en_pad = jnp.pad(en_col, ((0, _N_E_PAD - _N_E), (0, 0)),
                     constant_values=_PAD_VAL * _PAD_VAL * _E_DIM)
    idx3, loss_sum = _dist_argmin(latent, embedding, xs_row, en_pad)
    indices = idx3.reshape(-1)
    x_q = _sc_gather(embedding, indices)
    loss = loss_sum[0, 0] * (_BETA / (m * _E_DIM))
    return (x_q.reshape(x.shape), loss, indices.reshape(x.shape[:-1]))
